# baseline (speedup 1.0000x reference)
import functools

import jax
import jax.numpy as jnp
from jax import lax
from jax.experimental import pallas as pl
from jax.experimental.pallas import tpu as pltpu
from jax.experimental.pallas import tpu_sc as plsc

NN = 10000
DD = 128
NC = 2
NS = 16
NW = NC * NS
LL = 16

_BLK = 1000


@functools.lru_cache(maxsize=None)
def _sc_kernels(KB):
    mesh = plsc.VectorSubcoreMesh(
        core_axis_name="c", subcore_axis_name="s", num_cores=NC, num_subcores=NS
    )
    NPAD = 10240

    @functools.partial(
        pl.kernel, mesh=mesh,
        compiler_params=pltpu.CompilerParams(needs_layout_passes=False),
        out_type=jax.ShapeDtypeStruct((NW, NPAD), jnp.float32),
        scratch_types=[
            pltpu.VMEM((KB, 128), jnp.int32),
            pltpu.VMEM((NPAD,), jnp.float32),
        ],
    )
    def deg_kernel(dst_hbm, deg_hbm, didx, part):
        c = lax.axis_index("c")
        s = lax.axis_index("s")
        w = s * NC + c
        zeros16 = jnp.zeros((LL,), jnp.float32)
        ones16 = jnp.ones((LL,), jnp.float32)

        def zp(i, carry):
            part[pl.ds(i * LL, LL)] = zeros16
            return carry

        lax.fori_loop(0, NPAD // LL, zp, 0)
        pltpu.sync_copy(dst_hbm.at[pl.ds(w * KB, KB)], didx)

        def body(j, carry):
            for k in range(128 // LL):
                dv = didx.at[j][pl.ds(k * LL, LL)]
                plsc.addupdate_scatter(part, [dv], ones16)
            return carry

        lax.fori_loop(0, KB, body, 0)
        pltpu.sync_copy(part, deg_hbm.at[w])

    NACC = NN + 240
    ZRA = NACC // NS
    RPT = 624
    HD = DD // NC
    SB = 256
    KE = NC * KB * 128 // SB
    CH = 16
    NCH = KE // CH

    @functools.partial(
        pl.kernel, mesh=mesh,
        compiler_params=pltpu.CompilerParams(use_tc_tiling_on_sc=False),
        out_type=jax.ShapeDtypeStruct((NC, NN, HD), jnp.float32),
        scratch_types=[
            pltpu.VMEM((CH, SB), jnp.int32),
            pltpu.VMEM((CH, SB), jnp.int32),
            pltpu.VMEM((2, SB, HD), jnp.float32),
            pltpu.VMEM_SHARED((NN, HD), jnp.float32),
            pltpu.VMEM_SHARED((NACC, HD), jnp.float32),
            pltpu.SemaphoreType.DMA,
            pltpu.SemaphoreType.DMA,
            pltpu.SemaphoreType.DMA,
            pltpu.SemaphoreType.DMA,
        ],
    )
    def agg_kernel(hs_hbm, src_hbm, dst_hbm, out_hbm, sidx, didx, gbuf,
                   tab, acc, gsem0, gsem1, ssem0, ssem1):
        c = lax.axis_index("c")
        s = lax.axis_index("s")
        zeros16 = jnp.zeros((LL,), jnp.float32)
        gsems = (gsem0, gsem1)
        ssems = (ssem0, ssem1)

        def zb(i, carry):
            for k in range(HD // LL):
                gbuf[0, i, pl.ds(k * LL, LL)] = zeros16
            return carry

        lax.fori_loop(0, SB, zb, 0)
        base = s * ZRA
        for off in range(0, ZRA, SB):
            sz = min(SB, ZRA - off)
            pltpu.sync_copy(
                gbuf.at[0].at[pl.ds(0, sz)], acc.at[pl.ds(base + off, sz)]
            )
        rb = s * RPT
        last = (NS - 1) * RPT

        @pl.when(s < NS - 1)
        def _():
            pltpu.sync_copy(
                hs_hbm.at[c, pl.ds(rb, RPT)], tab.at[pl.ds(rb, RPT)]
            )

        @pl.when(s == NS - 1)
        def _():
            pltpu.sync_copy(
                hs_hbm.at[c, pl.ds(last, NN - last)],
                tab.at[pl.ds(last, NN - last)],
            )

        plsc.subcore_barrier()

        def chunk(ch, carry):
            jb = s * KE + ch * CH
            pltpu.sync_copy(src_hbm.at[pl.ds(jb, CH)], sidx)
            pltpu.sync_copy(dst_hbm.at[pl.ds(jb, CH)], didx)
            pltpu.async_copy(tab.at[sidx.at[0]], gbuf.at[0], gsem0)
            pltpu.async_copy(tab.at[sidx.at[1]], gbuf.at[1], gsem1)
            for rp in range(CH // 2):
                for b in range(2):
                    r = 2 * rp + b
                    pltpu.make_async_copy(
                        hs_hbm.at[c, pl.ds(0, SB)], gbuf.at[b], gsems[b]
                    ).wait()
                    pltpu.async_copy(
                        gbuf.at[b], acc.at[didx.at[r]], ssems[b], add=True
                    )
                if rp < CH // 2 - 1:
                    for b in range(2):
                        pltpu.make_async_copy(
                            hs_hbm.at[c, pl.ds(0, SB)], gbuf.at[b], ssems[b]
                        ).wait()
                        pltpu.async_copy(
                            tab.at[sidx.at[2 * (rp + 1) + b]], gbuf.at[b],
                            gsems[b],
                        )
            for b in range(2):
                pltpu.make_async_copy(
                    hs_hbm.at[c, pl.ds(0, SB)], gbuf.at[b], ssems[b]
                ).wait()
            return carry

        lax.fori_loop(0, NCH, chunk, 0)
        plsc.subcore_barrier()

        @pl.when(s < NS - 1)
        def _():
            pltpu.sync_copy(
                acc.at[pl.ds(rb, RPT)], out_hbm.at[c, pl.ds(rb, RPT)]
            )

        @pl.when(s == NS - 1)
        def _():
            last = (NS - 1) * RPT
            pltpu.sync_copy(
                acc.at[pl.ds(last, NN - last)],
                out_hbm.at[c, pl.ds(last, NN - last)],
            )

    return deg_kernel, agg_kernel


def _layer_norm(h, g, b):
    mu = jnp.mean(h, axis=-1, keepdims=True)
    var = jnp.mean((h - mu) ** 2, axis=-1, keepdims=True)
    return (h - mu) * lax.rsqrt(var + 1e-5) * g + b


def _dis(deg_ref):
    return lax.rsqrt(jnp.sum(deg_ref[...], axis=0) + 1.0)


def _split_store(o_ref, h):
    hd = DD // NC
    for i in range(NC):
        o_ref[i] = h[:, i * hd:(i + 1) * hd]


def _unsplit(ref):
    return jnp.concatenate([ref[i] for i in range(NC)], axis=-1)


def _tc_a(x_ref, w1_ref, deg_ref, o_ref):
    h = (
        jnp.dot(x_ref[...], w1_ref[...], preferred_element_type=jnp.float32)
        * _dis(deg_ref)
    )
    _split_store(o_ref, h)


def _tc_b(p_ref, hs_ref, deg_ref, b1_ref, g1_ref, be1_ref, w2_ref, o_ref):
    dis = _dis(deg_ref)
    t = (_unsplit(p_ref) + _unsplit(hs_ref)) * dis + b1_ref[...]
    h = _layer_norm(t, g1_ref[...], be1_ref[...])
    h = 0.5 * h * (1.0 + lax.erf(h * (2.0 ** -0.5)))
    h2 = jnp.dot(h, w2_ref[...], preferred_element_type=jnp.float32) * dis
    _split_store(o_ref, h2)


def _tc_c(q_ref, hs_ref, deg_ref, b2_ref, g2_ref, be2_ref, x_ref, o_ref):
    dis = _dis(deg_ref)
    t = (_unsplit(q_ref) + _unsplit(hs_ref)) * dis + b2_ref[...]
    h = _layer_norm(t, g2_ref[...], be2_ref[...])
    o = x_ref[...] + h
    nrm = jnp.sqrt(jnp.sum(o * o, axis=-1, keepdims=True))
    o_ref[...] = o / jnp.maximum(nrm, 1e-12)


def _row_spec():
    return pl.BlockSpec((_BLK, DD), lambda i: (i, 0))


def _full_spec():
    return pl.BlockSpec((DD, DD), lambda i: (0, 0))


def _vec_spec():
    return pl.BlockSpec((1, DD), lambda i: (0, 0))


def _deg_spec():
    return pl.BlockSpec((NW, _BLK, 1), lambda i: (0, i, 0))


def _pair_spec():
    return pl.BlockSpec((NC, _BLK, DD // NC), lambda i: (0, i, 0))


def kernel(x, edge_index, W1, b1, g1, be1, W2, b2, g2, be2):
    n, d = x.shape
    e = edge_index.shape[1]
    assert n == NN and d == DD
    KB = (-(-e // (NW * 128)) + 7) // 8 * 8
    pad = NW * KB * 128 - e
    src = jnp.concatenate([edge_index[0], jnp.zeros((pad,), jnp.int32)])
    dst = jnp.concatenate([edge_index[1], jnp.full((pad,), n, jnp.int32)])
    dst128 = dst.reshape(NW * KB, 128)
    src512 = src.reshape(-1, 256)
    dst512 = dst.reshape(-1, 256)

    deg_k, agg_k = _sc_kernels(KB)
    deg = deg_k(dst128)
    degc = deg[:, :n].reshape(NW, n, 1)

    b1r, g1r, be1r = b1.reshape(1, DD), g1.reshape(1, DD), be1.reshape(1, DD)
    b2r, g2r, be2r = b2.reshape(1, DD), g2.reshape(1, DD), be2.reshape(1, DD)
    grid = (n // _BLK,)
    row_shape = jax.ShapeDtypeStruct((n, DD), jnp.float32)
    pair_shape = jax.ShapeDtypeStruct((NC, n, DD // NC), jnp.float32)

    hs1 = pl.pallas_call(
        _tc_a,
        grid=grid,
        in_specs=[_row_spec(), _full_spec(), _deg_spec()],
        out_specs=_pair_spec(),
        out_shape=pair_shape,
    )(x, W1, degc)

    p = agg_k(hs1, src512, dst512)

    hs2 = pl.pallas_call(
        _tc_b,
        grid=grid,
        in_specs=[_pair_spec(), _pair_spec(), _deg_spec(), _vec_spec(),
                  _vec_spec(), _vec_spec(), _full_spec()],
        out_specs=_pair_spec(),
        out_shape=pair_shape,
    )(p, hs1, degc, b1r, g1r, be1r, W2)

    q = agg_k(hs2, src512, dst512)

    out = pl.pallas_call(
        _tc_c,
        grid=grid,
        in_specs=[_pair_spec(), _pair_spec(), _deg_spec(), _vec_spec(),
                  _vec_spec(), _vec_spec(), _row_spec()],
        out_specs=_row_spec(),
        out_shape=row_shape,
    )(q, hs2, degc, b2r, g2r, be2r, x)
    return out


# speedup vs baseline: 1.4212x; 1.4212x over previous
import functools

import jax
import jax.numpy as jnp
from jax import lax
from jax.experimental import pallas as pl
from jax.experimental.pallas import tpu as pltpu
from jax.experimental.pallas import tpu_sc as plsc

NN = 10000
DD = 128
NC = 2
NS = 16
NW = NC * NS
LL = 16

_BLK = 1000


@functools.lru_cache(maxsize=None)
def _sc_kernels(KB):
    mesh = plsc.VectorSubcoreMesh(
        core_axis_name="c", subcore_axis_name="s", num_cores=NC, num_subcores=NS
    )
    NPAD = 10240
    DSTR = NPAD // NS
    NACC = NN + 240
    ZRA = NACC // NS
    RPT = 624
    HD = DD // NC
    SB = 256
    KE = NC * KB * 128 // SB
    CH = 16
    NCH = KE // CH

    @functools.partial(
        pl.kernel, mesh=mesh,
        compiler_params=pltpu.CompilerParams(
            use_tc_tiling_on_sc=False, needs_layout_passes=False
        ),
        out_type=jax.ShapeDtypeStruct((NC, NN, HD), jnp.float32),
        scratch_types=[
            pltpu.VMEM((CH, SB), jnp.int32),
            pltpu.VMEM((CH, SB), jnp.int32),
            pltpu.VMEM((2, SB, HD), jnp.float32),
            pltpu.VMEM_SHARED((NN, HD), jnp.float32),
            pltpu.VMEM_SHARED((NACC, HD), jnp.float32),
            pltpu.SemaphoreType.DMA,
            pltpu.SemaphoreType.DMA,
            pltpu.SemaphoreType.DMA,
            pltpu.SemaphoreType.DMA,
        ],
    )
    def agg_kernel(hs_hbm, src_hbm, dst_hbm, out_hbm, sidx, didx, gbuf,
                   tab, acc, gsem0, gsem1, ssem0, ssem1):
        c = lax.axis_index("c")
        s = lax.axis_index("s")
        zeros16 = jnp.zeros((LL,), jnp.float32)
        gsems = (gsem0, gsem1)
        ssems = (ssem0, ssem1)

        def zb(i, carry):
            for k in range(HD // LL):
                gbuf[0, i, pl.ds(k * LL, LL)] = zeros16
            return carry

        lax.fori_loop(0, SB, zb, 0)
        base = s * ZRA
        for off in range(0, ZRA, SB):
            sz = min(SB, ZRA - off)
            pltpu.sync_copy(
                gbuf.at[0].at[pl.ds(0, sz)], acc.at[pl.ds(base + off, sz)]
            )
        rb = s * RPT
        last = (NS - 1) * RPT

        @pl.when(s < NS - 1)
        def _():
            pltpu.sync_copy(
                hs_hbm.at[c, pl.ds(rb, RPT)], tab.at[pl.ds(rb, RPT)]
            )

        @pl.when(s == NS - 1)
        def _():
            pltpu.sync_copy(
                hs_hbm.at[c, pl.ds(last, NN - last)],
                tab.at[pl.ds(last, NN - last)],
            )

        plsc.subcore_barrier()

        def chunk(ch, carry):
            jb = s * KE + ch * CH
            pltpu.sync_copy(src_hbm.at[pl.ds(jb, CH)], sidx)
            pltpu.sync_copy(dst_hbm.at[pl.ds(jb, CH)], didx)
            pltpu.async_copy(tab.at[sidx.at[0]], gbuf.at[0], gsem0)
            pltpu.async_copy(tab.at[sidx.at[1]], gbuf.at[1], gsem1)
            for rp in range(CH // 2):
                for b in range(2):
                    r = 2 * rp + b
                    pltpu.make_async_copy(
                        hs_hbm.at[c, pl.ds(0, SB)], gbuf.at[b], gsems[b]
                    ).wait()
                    pltpu.async_copy(
                        gbuf.at[b], acc.at[didx.at[r]], ssems[b], add=True
                    )
                if rp < CH // 2 - 1:
                    for b in range(2):
                        pltpu.make_async_copy(
                            hs_hbm.at[c, pl.ds(0, SB)], gbuf.at[b], ssems[b]
                        ).wait()
                        pltpu.async_copy(
                            tab.at[sidx.at[2 * (rp + 1) + b]], gbuf.at[b],
                            gsems[b],
                        )
            for b in range(2):
                pltpu.make_async_copy(
                    hs_hbm.at[c, pl.ds(0, SB)], gbuf.at[b], ssems[b]
                ).wait()
            return carry

        lax.fori_loop(0, NCH, chunk, 0)
        plsc.subcore_barrier()

        @pl.when(s < NS - 1)
        def _():
            pltpu.sync_copy(
                acc.at[pl.ds(rb, RPT)], out_hbm.at[c, pl.ds(rb, RPT)]
            )

        @pl.when(s == NS - 1)
        def _():
            last = (NS - 1) * RPT
            pltpu.sync_copy(
                acc.at[pl.ds(last, NN - last)],
                out_hbm.at[c, pl.ds(last, NN - last)],
            )

    C1 = 10
    NCH1 = KE // C1

    @functools.partial(
        pl.kernel, mesh=mesh,
        compiler_params=pltpu.CompilerParams(
            use_tc_tiling_on_sc=False, needs_layout_passes=False
        ),
        out_type=(
            jax.ShapeDtypeStruct((NC, NN, HD), jnp.float32),
            jax.ShapeDtypeStruct((NPAD,), jnp.float32),
            jax.ShapeDtypeStruct((NC, NS, NPAD), jnp.float32),
        ),
        scratch_types=[
            pltpu.VMEM((C1, SB), jnp.int32),
            pltpu.VMEM((C1, SB), jnp.int32),
            pltpu.VMEM((2, SB, HD), jnp.float32),
            pltpu.VMEM((NPAD,), jnp.float32),
            pltpu.VMEM_SHARED((NN, HD), jnp.float32),
            pltpu.VMEM_SHARED((NACC, HD), jnp.float32),
            pltpu.SemaphoreType.DMA,
            pltpu.SemaphoreType.DMA,
            pltpu.SemaphoreType.DMA,
            pltpu.SemaphoreType.DMA,
        ],
    )
    def deg_agg_kernel(x_hbm, src_hbm, dst_hbm, out_hbm, dis_hbm, parts_hbm,
                       sidx, didx, gbuf, part, tab, acc,
                       gsem0, gsem1, ssem0, ssem1):
        c = lax.axis_index("c")
        s = lax.axis_index("s")
        zeros16 = jnp.zeros((LL,), jnp.float32)
        ones16 = jnp.ones((LL,), jnp.float32)
        gsems = (gsem0, gsem1)
        ssems = (ssem0, ssem1)

        def zb(i, carry):
            for k in range(HD // LL):
                gbuf[0, i, pl.ds(k * LL, LL)] = zeros16
            return carry

        lax.fori_loop(0, SB, zb, 0)
        base = s * ZRA
        for off in range(0, ZRA, SB):
            sz = min(SB, ZRA - off)
            pltpu.sync_copy(
                gbuf.at[0].at[pl.ds(0, sz)], acc.at[pl.ds(base + off, sz)]
            )

        def zp(i, carry):
            part[pl.ds(i * LL, LL)] = zeros16
            return carry

        lax.fori_loop(0, NPAD // LL, zp, 0)

        def dchunk(ch, carry):
            pltpu.sync_copy(
                dst_hbm.at[pl.ds(s * KE + ch * C1, C1)], didx
            )
            for r in range(C1):
                for k in range(SB // LL):
                    dv = didx.at[r][pl.ds(k * LL, LL)]
                    plsc.addupdate_scatter(part, [dv], ones16)
            return carry

        lax.fori_loop(0, NCH1, dchunk, 0)
        pltpu.sync_copy(part, parts_hbm.at[c, s])
        plsc.subcore_barrier()

        for k in range(NS):
            pltpu.sync_copy(
                parts_hbm.at[c, k, pl.ds(s * DSTR, DSTR)],
                part.at[pl.ds(k * DSTR, DSTR)],
            )

        def disv(i, carry):
            v = part[pl.ds(i * LL, LL)]
            for k in range(1, NS):
                v = v + part[pl.ds(k * DSTR + i * LL, LL)]
            v = v + 1.0
            y = plsc.bitcast(
                0x5F3759DF - lax.shift_right_logical(
                    plsc.bitcast(v, jnp.int32), 1
                ),
                jnp.float32,
            )
            h = v * -0.5
            for _ in range(3):
                y = y * (1.5 + h * y * y)
            part[pl.ds(i * LL, LL)] = y
            return carry

        lax.fori_loop(0, DSTR // LL, disv, 0)

        @pl.when(c == 0)
        def _():
            pltpu.sync_copy(
                part.at[pl.ds(0, DSTR)], dis_hbm.at[pl.ds(s * DSTR, DSTR)]
            )

        def stage(nrows):
            for off in range(0, nrows, SB):
                sz = min(SB, nrows - off)
                pltpu.sync_copy(
                    x_hbm.at[pl.ds(s * DSTR + off, sz), pl.ds(c * HD, HD)],
                    gbuf.at[0].at[pl.ds(0, sz)],
                )

                def scale(g, carry):
                    dvec = part[pl.ds(off + g * LL, LL)]
                    for lane in range(LL):
                        d = dvec[lane]
                        row = g * LL + lane
                        for k in range(HD // LL):
                            gbuf[0, row, pl.ds(k * LL, LL)] = (
                                gbuf[0, row, pl.ds(k * LL, LL)] * d
                            )
                    return carry

                lax.fori_loop(0, sz // LL, scale, 0)
                pltpu.sync_copy(
                    gbuf.at[0].at[pl.ds(0, sz)],
                    tab.at[pl.ds(s * DSTR + off, sz)],
                )

        @pl.when(s < NS - 1)
        def _():
            stage(DSTR)

        @pl.when(s == NS - 1)
        def _():
            stage(NN - (NS - 1) * DSTR)

        plsc.subcore_barrier()

        def chunk(ch, carry):
            jb = s * KE + ch * C1
            pltpu.sync_copy(src_hbm.at[pl.ds(jb, C1)], sidx)
            pltpu.sync_copy(dst_hbm.at[pl.ds(jb, C1)], didx)
            pltpu.async_copy(tab.at[sidx.at[0]], gbuf.at[0], gsem0)
            pltpu.async_copy(tab.at[sidx.at[1]], gbuf.at[1], gsem1)
            for rp in range(C1 // 2):
                for b in range(2):
                    r = 2 * rp + b
                    pltpu.make_async_copy(
                        x_hbm.at[pl.ds(0, SB), pl.ds(0, HD)], gbuf.at[b],
                        gsems[b],
                    ).wait()
                    pltpu.async_copy(
                        gbuf.at[b], acc.at[didx.at[r]], ssems[b], add=True
                    )
                if rp < C1 // 2 - 1:
                    for b in range(2):
                        pltpu.make_async_copy(
                            x_hbm.at[pl.ds(0, SB), pl.ds(0, HD)], gbuf.at[b],
                            ssems[b],
                        ).wait()
                        pltpu.async_copy(
                            tab.at[sidx.at[2 * (rp + 1) + b]], gbuf.at[b],
                            gsems[b],
                        )
            for b in range(2):
                pltpu.make_async_copy(
                    x_hbm.at[pl.ds(0, SB), pl.ds(0, HD)], gbuf.at[b],
                    ssems[b],
                ).wait()
            return carry

        lax.fori_loop(0, NCH1, chunk, 0)
        plsc.subcore_barrier()
        rb1 = s * RPT
        last1 = (NS - 1) * RPT

        @pl.when(s < NS - 1)
        def _():
            pltpu.sync_copy(
                acc.at[pl.ds(rb1, RPT)], out_hbm.at[c, pl.ds(rb1, RPT)]
            )

        @pl.when(s == NS - 1)
        def _():
            pltpu.sync_copy(
                acc.at[pl.ds(last1, NN - last1)],
                out_hbm.at[c, pl.ds(last1, NN - last1)],
            )

    return deg_agg_kernel, agg_kernel


def _layer_norm(h, g, b):
    mu = jnp.mean(h, axis=-1, keepdims=True)
    var = jnp.mean((h - mu) ** 2, axis=-1, keepdims=True)
    return (h - mu) * lax.rsqrt(var + 1e-5) * g + b


def _split_store(o_ref, h):
    hd = DD // NC
    for i in range(NC):
        o_ref[i] = h[:, i * hd:(i + 1) * hd]


def _unsplit(ref):
    return jnp.concatenate([ref[i] for i in range(NC)], axis=-1)


def _tc_b(p_ref, x_ref, dis_ref, b1_ref, g1_ref, be1_ref, w1_ref, o_ref):
    dis = dis_ref[...]
    t = (_unsplit(p_ref) + x_ref[...] * dis) * dis
    pre = (
        jnp.dot(t, w1_ref[...], preferred_element_type=jnp.float32)
        + b1_ref[...]
    )
    h = _layer_norm(pre, g1_ref[...], be1_ref[...])
    h = 0.5 * h * (1.0 + lax.erf(h * (2.0 ** -0.5)))
    _split_store(o_ref, h * dis)


def _tc_c(q_ref, hs_ref, dis_ref, b2_ref, g2_ref, be2_ref, w2_ref, x_ref,
          o_ref):
    dis = dis_ref[...]
    t = (_unsplit(q_ref) + _unsplit(hs_ref)) * dis
    pre = (
        jnp.dot(t, w2_ref[...], preferred_element_type=jnp.float32)
        + b2_ref[...]
    )
    h = _layer_norm(pre, g2_ref[...], be2_ref[...])
    o = x_ref[...] + h
    nrm = jnp.sqrt(jnp.sum(o * o, axis=-1, keepdims=True))
    o_ref[...] = o / jnp.maximum(nrm, 1e-12)


def _row_spec():
    return pl.BlockSpec((_BLK, DD), lambda i: (i, 0))


def _full_spec():
    return pl.BlockSpec((DD, DD), lambda i: (0, 0))


def _vec_spec():
    return pl.BlockSpec((1, DD), lambda i: (0, 0))


def _dis_spec():
    return pl.BlockSpec((_BLK, 1), lambda i: (i, 0))


def _pair_spec():
    return pl.BlockSpec((NC, _BLK, DD // NC), lambda i: (0, i, 0))


def kernel(x, edge_index, W1, b1, g1, be1, W2, b2, g2, be2):
    n, d = x.shape
    e = edge_index.shape[1]
    assert n == NN and d == DD
    KB = (-(-e // (NW * 128)) + 7) // 8 * 8
    pad = NW * KB * 128 - e
    src = jnp.concatenate([edge_index[0], jnp.zeros((pad,), jnp.int32)])
    dst = jnp.concatenate([edge_index[1], jnp.full((pad,), n, jnp.int32)])
    src256 = src.reshape(-1, 256)
    dst256 = dst.reshape(-1, 256)

    sc1_k, agg_k = _sc_kernels(KB)
    p, dis, _parts = sc1_k(x, src256, dst256)
    disc = dis[:n].reshape(n, 1)

    b1r, g1r, be1r = b1.reshape(1, DD), g1.reshape(1, DD), be1.reshape(1, DD)
    b2r, g2r, be2r = b2.reshape(1, DD), g2.reshape(1, DD), be2.reshape(1, DD)
    grid = (n // _BLK,)
    row_shape = jax.ShapeDtypeStruct((n, DD), jnp.float32)
    pair_shape = jax.ShapeDtypeStruct((NC, n, DD // NC), jnp.float32)

    hs2 = pl.pallas_call(
        _tc_b,
        grid=grid,
        in_specs=[_pair_spec(), _row_spec(), _dis_spec(), _vec_spec(),
                  _vec_spec(), _vec_spec(), _full_spec()],
        out_specs=_pair_spec(),
        out_shape=pair_shape,
    )(p, x, disc, b1r, g1r, be1r, W1)

    q = agg_k(hs2, src256, dst256)

    out = pl.pallas_call(
        _tc_c,
        grid=grid,
        in_specs=[_pair_spec(), _pair_spec(), _dis_spec(), _vec_spec(),
                  _vec_spec(), _vec_spec(), _full_spec(), _row_spec()],
        out_specs=_row_spec(),
        out_shape=row_shape,
    )(q, hs2, disc, b2r, g2r, be2r, W2, x)
    return out
